# baseline (speedup 1.0000x reference)
import functools

import jax
import jax.numpy as jnp
from jax import lax
from jax.experimental import pallas as pl
from jax.experimental.pallas import tpu as pltpu
from jax.experimental.pallas import tpu_sc as plsc

CHUNK = 128
NBUF = 6
AHEAD = 4


def _build(B, D, n_chunks_total):
    info = plsc.get_sparse_core_info()
    NC, NS = info.num_cores, info.num_subcores
    NW = NC * NS
    assert n_chunks_total % NW == 0
    chunks_per_w = n_chunks_total // NW
    rows_per_w = chunks_per_w * CHUNK
    scale = float(D) ** 0.5

    mesh = plsc.VectorSubcoreMesh(core_axis_name="c", subcore_axis_name="s")

    @functools.partial(
        pl.kernel,
        out_type=jax.ShapeDtypeStruct((B, D), jnp.float32),
        mesh=mesh,
        compiler_params=pltpu.CompilerParams(use_tc_tiling_on_sc=False),
        scratch_types=[
            pltpu.VMEM((chunks_per_w, CHUNK), jnp.int32),
            pltpu.VMEM((NBUF, CHUNK, D), jnp.float32),
            pltpu.SemaphoreType.DMA((NBUF,)),
            pltpu.SemaphoreType.DMA((NBUF,)),
        ],
    )
    def emb_kernel(table_hbm, idx_hbm, out_hbm, idx_v, buf, gsem, osem):
        wid = lax.axis_index("s") * NC + lax.axis_index("c")
        row_base = wid * rows_per_w
        chunk_base = wid * chunks_per_w

        pltpu.sync_copy(idx_hbm.at[pl.ds(chunk_base, chunks_per_w)], idx_v)

        def fire_gather(c, p):
            pltpu.async_copy(table_hbm.at[idx_v.at[c]], buf.at[p], gsem.at[p])

        def wait_gather(c, p):
            pltpu.make_async_copy(
                table_hbm.at[idx_v.at[c]], buf.at[p], gsem.at[p]
            ).wait()

        def out_slice(c):
            return out_hbm.at[pl.ds(row_base + c * CHUNK, CHUNK)]

        for c in range(AHEAD):
            fire_gather(c, c)

        def body(g, carry):
            p = lax.rem(g, NBUF)
            wait_gather(g, p)

            def mul_row(r, carry2):
                for cc in range(D // 16):
                    sl = pl.ds(cc * 16, 16)
                    buf[p, r, sl] = buf[p, r, sl] * scale
                return carry2

            lax.fori_loop(0, CHUNK, mul_row, 0, unroll=2)

            pltpu.async_copy(buf.at[p], out_slice(g), osem.at[p])

            @pl.when(g + AHEAD < chunks_per_w)
            def _():
                pn = lax.rem(g + AHEAD, NBUF)

                @pl.when(g >= NBUF - AHEAD)
                def _():
                    pltpu.make_async_copy(
                        buf.at[pn], out_slice(g), osem.at[pn]
                    ).wait()

                fire_gather(g + AHEAD, pn)

            return carry

        lax.fori_loop(0, chunks_per_w, body, 0)

        for p in range(NBUF):
            pltpu.make_async_copy(buf.at[p], out_slice(0), osem.at[p]).wait()

    return emb_kernel


def kernel(X, weight):
    batch, hist = X.shape
    vocab, d = weight.shape
    n_total = batch * hist
    idx = X.reshape(n_total // CHUNK, CHUNK).astype(jnp.int32)
    emb_kernel = _build(n_total, d, n_total // CHUNK)
    out = emb_kernel(weight, idx)
    return out.reshape(batch, hist, d)
